# Initial kernel scaffold; baseline (speedup 1.0000x reference)
import jax
import jax.numpy as jnp
from jax.experimental import pallas as pl


def kernel(sequence, segment_label, tok_table, seg_table):
    raise NotImplementedError("write your pallas kernel here")


# speedup vs baseline: 1.2262x; 1.2262x over previous
import functools

import numpy as np
import jax
import jax.numpy as jnp
from jax import lax
from jax.experimental import pallas as pl
from jax.experimental.pallas import tpu as pltpu
from jax.experimental.pallas import tpu_sc as plsc

VOCAB = 1000000
D = 64
B = 1024
S = 200

NC = 2
NS = 16
NW = NC * NS
TOTAL = B * S
PER_W = TOTAL // NW
CHUNK = 256
KIDX = CHUNK // 128
NCHUNK = PER_W // CHUNK


def _make_pe():
    pos = np.arange(S, dtype=np.float32)[:, None]
    div = np.exp(np.arange(0, D, 2, dtype=np.float32) * -(np.log(10000.0) / D))
    pe = np.zeros((S, D), dtype=np.float32)
    pe[:, 0::2] = np.sin(pos * div)
    pe[:, 1::2] = np.cos(pos * div)
    return pe


_PE = _make_pe()


def _comb_body(pe_ref, seg_ref, out_ref):
    pe = pe_ref[...]
    for l in range(3):
        out_ref[:, l * D:(l + 1) * D] = pe + seg_ref[l, :][None, :]


def _build_comb(seg_table):
    comb2 = pl.pallas_call(
        _comb_body,
        out_shape=jax.ShapeDtypeStruct((S, 3 * D), jnp.float32),
    )(_PE, seg_table)
    return comb2.reshape(3 * S, D)


_mesh = plsc.VectorSubcoreMesh(core_axis_name="c", subcore_axis_name="s")


@functools.partial(
    pl.kernel,
    mesh=_mesh,
    out_type=jax.ShapeDtypeStruct((TOTAL, D), jnp.float32),
    scratch_types=[
        pltpu.VMEM((CHUNK,), jnp.int32),
        pltpu.VMEM((CHUNK,), jnp.int32),
        pltpu.VMEM((CHUNK, D), jnp.float32),
        pltpu.VMEM((CHUNK, D), jnp.float32),
        pltpu.SemaphoreType.DMA,
        pltpu.SemaphoreType.DMA,
    ],
    compiler_params=pltpu.CompilerParams(use_tc_tiling_on_sc=False),
)
def _emb(seq_hbm, lab_hbm, tok_hbm, comb_hbm, out_hbm,
         tok_idx, cmb_idx, tok_v, cmb_v, sem_t, sem_c):
    wid = lax.axis_index("s") * NC + lax.axis_index("c")
    w0 = wid * PER_W

    def chunk(c, carry):
        base = pl.multiple_of(w0 + c * CHUNK, CHUNK)
        pltpu.sync_copy(seq_hbm.at[pl.ds(base, CHUNK)], tok_idx)
        pltpu.sync_copy(lab_hbm.at[pl.ds(base, CHUNK)], cmb_idx)
        for k in range(CHUNK // 16):
            sl = pl.ds(k * 16, 16)
            lab = cmb_idx[sl]
            v = base + k * 16 + lax.iota(jnp.int32, 16)
            cmb_idx[sl] = lax.rem(v, S) * 3 + lab
        cps = []
        for jj in range(KIDX):
            cps.append(pltpu.async_copy(
                tok_hbm.at[tok_idx.at[pl.ds(jj * 128, 128)]],
                tok_v.at[pl.ds(jj * 128, 128)], sem_t))
            cps.append(pltpu.async_copy(
                comb_hbm.at[cmb_idx.at[pl.ds(jj * 128, 128)]],
                cmb_v.at[pl.ds(jj * 128, 128)], sem_c))
        for cp in cps:
            cp.wait()

        def add_row(i, carry2):
            for k in range(D // 16):
                sl = pl.ds(k * 16, 16)
                tok_v[i, sl] = tok_v[i, sl] + cmb_v[i, sl]
            return carry2

        lax.fori_loop(0, CHUNK, add_row, 0)
        pltpu.sync_copy(tok_v, out_hbm.at[pl.ds(base, CHUNK)])
        return carry

    lax.fori_loop(0, NCHUNK, chunk, 0)


def kernel(sequence, segment_label, tok_table, seg_table):
    comb = _build_comb(seg_table)
    seq = sequence.reshape(TOTAL)
    lab = segment_label.reshape(TOTAL)
    out = _emb(seq, lab, tok_table, comb)
    return out.reshape(B, S, D)
